# baseline (speedup 1.0000x reference)
import functools

import jax
import jax.numpy as jnp
from jax import lax
from jax.experimental import pallas as pl
from jax.experimental.pallas import tpu as pltpu
from jax.experimental.pallas import tpu_sc as plsc

_EMB_SIZE = 1000000
_EMB_DIM = 64
_B = 16384
_CTX = 20
_NC = 2
_NS = 16
_NW = _NC * _NS
_BPW = _B // _NW
_E = 16
_CHUNKS = _BPW // _E

_LN2 = 0.6931471805599453
_SQRT2 = 1.4142135623730951

_DNUMS = lax.GatherDimensionNumbers(
    offset_dims=(), collapsed_slice_dims=(0,), start_index_map=(0,))


def _log_pos(a):
    i = lax.bitcast_convert_type(a, jnp.int32)
    e = lax.shift_right_arithmetic(i, 23) - 127
    m = lax.bitcast_convert_type(
        jnp.bitwise_or(jnp.bitwise_and(i, 0x007FFFFF), 0x3F800000), jnp.float32)
    big = m > _SQRT2
    m = jnp.where(big, m * 0.5, m)
    e = jnp.where(big, e + 1, e)
    t = (m - 1.0) / (m + 1.0)
    t2 = t * t
    series = 1.0 + t2 * (1.0 / 3.0 + t2 * (1.0 / 5.0 + t2 * (
        1.0 / 7.0 + t2 * (1.0 / 9.0 + t2 * (1.0 / 11.0)))))
    return e.astype(jnp.float32) * _LN2 + 2.0 * t * series


def _lane_sum_splat(v):
    idx = jnp.arange(16, dtype=jnp.int32)
    for s in (1, 2, 4, 8):
        perm = jnp.bitwise_xor(idx, s)
        v = v + lax.gather(v, perm[:, None], dimension_numbers=_DNUMS,
                           slice_sizes=(1,),
                           mode=lax.GatherScatterMode.PROMISE_IN_BOUNDS)
    return v


def _logsigmoid(x):
    return -_log_pos(1.0 + jnp.exp(-x))


def _sc_body(w_ctx, w_word, ctx_t, pw_idx, neg_t, out,
             idx_ctx_o, idx_neg_o, idx_pw_o,
             phys_c, phys_n, phys_p,
             slab_ctx, slab_neg, rows_pw, out_v, sem):
    wid = lax.axis_index("s") * _NC + lax.axis_index("c")

    for j in range(_CTX):
        pltpu.sync_copy(ctx_t.at[pl.ds(j * _B + wid * _BPW, _BPW)],
                        idx_ctx_o.at[j])
        pltpu.sync_copy(neg_t.at[pl.ds(j * _B + wid * _BPW, _BPW)],
                        idx_neg_o.at[j])
    pltpu.sync_copy(pw_idx.at[pl.ds(wid * _BPW, _BPW)], idx_pw_o)

    def chunk_body(c, acc):
        e0 = c * _E
        for j in range(_CTX):
            phys_c[pl.ds(j * _E, 16)] = idx_ctx_o[j, pl.ds(e0, 16)]
            phys_n[pl.ds(j * _E, 16)] = idx_neg_o[j, pl.ds(e0, 16)]
        phys_p[...] = idx_pw_o[pl.ds(e0, 16)]
        copies = []
        for start, size in ((0, 128), (128, 128), (256, 64)):
            copies.append(pltpu.async_copy(
                w_ctx.at[phys_c.at[pl.ds(start, size)]],
                slab_ctx.at[pl.ds(start, size)], sem))
            copies.append(pltpu.async_copy(
                w_word.at[phys_n.at[pl.ds(start, size)]],
                slab_neg.at[pl.ds(start, size)], sem))
        copies.append(pltpu.async_copy(w_word.at[phys_p], rows_pw, sem))
        for cp in copies:
            cp.wait()

        def ex_body(e, acc2):
            a = [slab_ctx[e, pl.ds(dc * 16, 16)] for dc in range(4)]
            nacc = [slab_neg[e, pl.ds(dc * 16, 16)] for dc in range(4)]
            for j in range(1, _CTX):
                r = j * _E + e
                for dc in range(4):
                    a[dc] = a[dc] + slab_ctx[r, pl.ds(dc * 16, 16)]
                    nacc[dc] = nacc[dc] + slab_neg[r, pl.ds(dc * 16, 16)]
            pvec = [rows_pw[e, pl.ds(dc * 16, 16)] for dc in range(4)]
            sp = a[0] * pvec[0] + a[1] * pvec[1] + a[2] * pvec[2] + a[3] * pvec[3]
            sn = a[0] * nacc[0] + a[1] * nacc[1] + a[2] * nacc[2] + a[3] * nacc[3]
            spl_sp = _lane_sum_splat(sp)
            spl_sn = _lane_sum_splat(sn)
            x = jnp.where(jnp.arange(16, dtype=jnp.int32) < 8, spl_sp, -spl_sn)
            return acc2 + _logsigmoid(x)

        return lax.fori_loop(0, _E, ex_body, acc, unroll=2)

    accv = lax.fori_loop(0, _CHUNKS, chunk_body,
                         jnp.zeros((16,), jnp.float32), unroll=False)
    total = _lane_sum_splat(accv) * 0.125
    out_v[0, :] = jnp.where(jnp.arange(16, dtype=jnp.int32) == 0, total, 0.0)
    zeros = jnp.zeros((16,), jnp.float32)
    for r in range(1, 8):
        out_v[r, :] = zeros
    pltpu.sync_copy(out_v, out.at[wid])


@jax.jit
def _cbow_loss_sc(w_ctx, w_word, ctx_t, pw_idx, neg_t):
    mesh = plsc.VectorSubcoreMesh(core_axis_name="c", subcore_axis_name="s")
    kfn = functools.partial(
        pl.kernel, mesh=mesh,
        out_type=jax.ShapeDtypeStruct((_NW, 8, 16), jnp.float32),
        scratch_types=[
            pltpu.VMEM((_CTX, _BPW), jnp.int32),
            pltpu.VMEM((_CTX, _BPW), jnp.int32),
            pltpu.VMEM((_BPW,), jnp.int32),
            pltpu.VMEM((_CTX * _E,), jnp.int32),
            pltpu.VMEM((_CTX * _E,), jnp.int32),
            pltpu.VMEM((_E,), jnp.int32),
            pltpu.VMEM((_CTX * _E, 2 * _EMB_DIM), jnp.float32),
            pltpu.VMEM((_CTX * _E, 2 * _EMB_DIM), jnp.float32),
            pltpu.VMEM((_E, 2 * _EMB_DIM), jnp.float32),
            pltpu.VMEM((8, 16), jnp.float32),
            pltpu.SemaphoreType.DMA,
        ],
    )(_sc_body)
    return kfn(w_ctx, w_word, ctx_t, pw_idx, neg_t)


def kernel(W_ctx, W_word, pos_context, pos_word, neg_word):
    proj = jnp.eye(_EMB_DIM, 2 * _EMB_DIM, dtype=jnp.float32)
    w_ctx2 = jnp.dot(W_ctx, proj)
    w_word2 = jnp.dot(W_word, proj)
    ctx_t = pos_context.T.reshape(-1)
    neg_t = neg_word.T.reshape(-1)
    partials = _cbow_loss_sc(w_ctx2, w_word2, ctx_t, pos_word, neg_t)
    return -jnp.sum(partials)


# speedup vs baseline: 1.0990x; 1.0990x over previous
import functools

import jax
import jax.numpy as jnp
from jax import lax
from jax.experimental import pallas as pl
from jax.experimental.pallas import tpu as pltpu
from jax.experimental.pallas import tpu_sc as plsc

_EMB_SIZE = 1000000
_EMB_DIM = 64
_B = 16384
_CTX = 20
_NC = 2
_NS = 16
_NW = _NC * _NS
_BPW = _B // _NW
_E = 16
_CHUNKS = _BPW // _E

_LN2 = 0.6931471805599453
_SQRT2 = 1.4142135623730951

_DNUMS = lax.GatherDimensionNumbers(
    offset_dims=(), collapsed_slice_dims=(0,), start_index_map=(0,))


def _log_pos(a):
    i = lax.bitcast_convert_type(a, jnp.int32)
    e = lax.shift_right_arithmetic(i, 23) - 127
    m = lax.bitcast_convert_type(
        jnp.bitwise_or(jnp.bitwise_and(i, 0x007FFFFF), 0x3F800000), jnp.float32)
    big = m > _SQRT2
    m = jnp.where(big, m * 0.5, m)
    e = jnp.where(big, e + 1, e)
    t = (m - 1.0) / (m + 1.0)
    t2 = t * t
    series = 1.0 + t2 * (1.0 / 3.0 + t2 * (1.0 / 5.0 + t2 * (
        1.0 / 7.0 + t2 * (1.0 / 9.0 + t2 * (1.0 / 11.0)))))
    return e.astype(jnp.float32) * _LN2 + 2.0 * t * series


def _lane_sum_splat(v):
    idx = jnp.arange(16, dtype=jnp.int32)
    for s in (1, 2, 4, 8):
        perm = jnp.bitwise_xor(idx, s)
        v = v + lax.gather(v, perm[:, None], dimension_numbers=_DNUMS,
                           slice_sizes=(1,),
                           mode=lax.GatherScatterMode.PROMISE_IN_BOUNDS)
    return v


def _logsigmoid(x):
    return -_log_pos(1.0 + jnp.exp(-x))


def _sc_body(w_ctx, w_word, ctx_t, pw_idx, neg_t, out,
             idx_ctx_o, idx_neg_o, idx_pw_o,
             phys_c, phys_n, phys_p,
             slab_ctx, slab_neg, rows_pw, out_v, sem):
    wid = lax.axis_index("s") * _NC + lax.axis_index("c")

    for j in range(_CTX):
        pltpu.sync_copy(ctx_t.at[pl.ds(j * _B + wid * _BPW, _BPW)],
                        idx_ctx_o.at[j])
        pltpu.sync_copy(neg_t.at[pl.ds(j * _B + wid * _BPW, _BPW)],
                        idx_neg_o.at[j])
    pltpu.sync_copy(pw_idx.at[pl.ds(wid * _BPW, _BPW)], idx_pw_o)

    def chunk_body(c, acc):
        e0 = c * _E
        for j in range(_CTX):
            phys_c[pl.ds(j * _E, 16)] = idx_ctx_o[j, pl.ds(e0, 16)]
            phys_n[pl.ds(j * _E, 16)] = idx_neg_o[j, pl.ds(e0, 16)]
        phys_p[...] = idx_pw_o[pl.ds(e0, 16)]
        copies = []
        for start, size in ((0, 128), (128, 128), (256, 64)):
            copies.append(pltpu.async_copy(
                w_ctx.at[phys_c.at[pl.ds(start, size)]],
                slab_ctx.at[pl.ds(start, size)], sem))
            copies.append(pltpu.async_copy(
                w_word.at[phys_n.at[pl.ds(start, size)]],
                slab_neg.at[pl.ds(start, size)], sem))
        copies.append(pltpu.async_copy(w_word.at[phys_p], rows_pw, sem))
        for cp in copies:
            cp.wait()

        def ex_body(e, acc2):
            a = [slab_ctx[e, pl.ds(dc * 16, 16)] for dc in range(4)]
            nacc = [slab_neg[e, pl.ds(dc * 16, 16)] for dc in range(4)]
            for j in range(1, _CTX):
                r = j * _E + e
                for dc in range(4):
                    a[dc] = a[dc] + slab_ctx[r, pl.ds(dc * 16, 16)]
                    nacc[dc] = nacc[dc] + slab_neg[r, pl.ds(dc * 16, 16)]
            pvec = [rows_pw[e, pl.ds(dc * 16, 16)] for dc in range(4)]
            sp = a[0] * pvec[0] + a[1] * pvec[1] + a[2] * pvec[2] + a[3] * pvec[3]
            sn = a[0] * nacc[0] + a[1] * nacc[1] + a[2] * nacc[2] + a[3] * nacc[3]
            spl_sp = _lane_sum_splat(sp)
            spl_sn = _lane_sum_splat(sn)
            x = jnp.where(jnp.arange(16, dtype=jnp.int32) < 8, spl_sp, -spl_sn)
            return acc2 + _logsigmoid(x)

        return lax.fori_loop(0, _E, ex_body, acc, unroll=False)

    accv = lax.fori_loop(0, _CHUNKS, chunk_body,
                         jnp.zeros((16,), jnp.float32), unroll=False)
    total = _lane_sum_splat(accv) * 0.125
    out_v[0, :] = jnp.where(jnp.arange(16, dtype=jnp.int32) == 0, total, 0.0)
    zeros = jnp.zeros((16,), jnp.float32)
    for r in range(1, 8):
        out_v[r, :] = zeros
    pltpu.sync_copy(out_v, out.at[wid])


@jax.jit
def _cbow_loss_sc(w_ctx, w_word, ctx_t, pw_idx, neg_t):
    mesh = plsc.VectorSubcoreMesh(core_axis_name="c", subcore_axis_name="s")
    kfn = functools.partial(
        pl.kernel, mesh=mesh,
        out_type=jax.ShapeDtypeStruct((_NW, 8, 16), jnp.float32),
        scratch_types=[
            pltpu.VMEM((_CTX, _BPW), jnp.int32),
            pltpu.VMEM((_CTX, _BPW), jnp.int32),
            pltpu.VMEM((_BPW,), jnp.int32),
            pltpu.VMEM((_CTX * _E,), jnp.int32),
            pltpu.VMEM((_CTX * _E,), jnp.int32),
            pltpu.VMEM((_E,), jnp.int32),
            pltpu.VMEM((_CTX * _E, 2 * _EMB_DIM), jnp.float32),
            pltpu.VMEM((_CTX * _E, 2 * _EMB_DIM), jnp.float32),
            pltpu.VMEM((_E, 2 * _EMB_DIM), jnp.float32),
            pltpu.VMEM((8, 16), jnp.float32),
            pltpu.SemaphoreType.DMA,
        ],
    )(_sc_body)
    return kfn(w_ctx, w_word, ctx_t, pw_idx, neg_t)


def kernel(W_ctx, W_word, pos_context, pos_word, neg_word):
    proj = jnp.eye(_EMB_DIM, 2 * _EMB_DIM, dtype=jnp.float32)
    w_ctx2 = jnp.dot(W_ctx, proj)
    w_word2 = jnp.dot(W_word, proj)
    ctx_t = pos_context.T.reshape(-1)
    neg_t = neg_word.T.reshape(-1)
    partials = _cbow_loss_sc(w_ctx2, w_word2, ctx_t, pos_word, neg_t)
    return -jnp.sum(partials)


# speedup vs baseline: 1.1305x; 1.0287x over previous
import functools

import jax
import jax.numpy as jnp
from jax import lax
from jax.experimental import pallas as pl
from jax.experimental.pallas import tpu as pltpu
from jax.experimental.pallas import tpu_sc as plsc

_EMB_SIZE = 1000000
_EMB_DIM = 64
_B = 16384
_CTX = 20
_NC = 2
_NS = 16
_NW = _NC * _NS
_BPW = _B // _NW
_E = 16
_CHUNKS = _BPW // _E

_LN2 = 0.6931471805599453
_SQRT2 = 1.4142135623730951

_DNUMS = lax.GatherDimensionNumbers(
    offset_dims=(), collapsed_slice_dims=(0,), start_index_map=(0,))


def _log_pos(a):
    i = lax.bitcast_convert_type(a, jnp.int32)
    e = lax.shift_right_arithmetic(i, 23) - 127
    m = lax.bitcast_convert_type(
        jnp.bitwise_or(jnp.bitwise_and(i, 0x007FFFFF), 0x3F800000), jnp.float32)
    big = m > _SQRT2
    m = jnp.where(big, m * 0.5, m)
    e = jnp.where(big, e + 1, e)
    t = (m - 1.0) / (m + 1.0)
    t2 = t * t
    series = 1.0 + t2 * (1.0 / 3.0 + t2 * (1.0 / 5.0 + t2 * (
        1.0 / 7.0 + t2 * (1.0 / 9.0 + t2 * (1.0 / 11.0)))))
    return e.astype(jnp.float32) * _LN2 + 2.0 * t * series


def _lane_sum_splat(v):
    idx = jnp.arange(16, dtype=jnp.int32)
    for s in (1, 2, 4, 8):
        perm = jnp.bitwise_xor(idx, s)
        v = v + lax.gather(v, perm[:, None], dimension_numbers=_DNUMS,
                           slice_sizes=(1,),
                           mode=lax.GatherScatterMode.PROMISE_IN_BOUNDS)
    return v


def _logsigmoid(x):
    return -_log_pos(1.0 + jnp.exp(-x))


def _sc_ctx_body(w_ctx, ctx_t, a_out,
                 idx_ctx_o, phys_c, slab_ctx, a_buf, sem):
    wid = lax.axis_index("s") * _NC + lax.axis_index("c")

    for j in range(_CTX):
        pltpu.sync_copy(ctx_t.at[pl.ds(j * _B + wid * _BPW, _BPW)],
                        idx_ctx_o.at[j])

    def chunk_body(c, carry):
        e0 = c * _E
        for j in range(_CTX):
            phys_c[pl.ds(j * _E, 16)] = idx_ctx_o[j, pl.ds(e0, 16)]
        copies = []
        for start, size in ((0, 128), (128, 128), (256, 64)):
            copies.append(pltpu.async_copy(
                w_ctx.at[phys_c.at[pl.ds(start, size)]],
                slab_ctx.at[pl.ds(start, size)], sem))
        for cp in copies:
            cp.wait()

        def ex_pair_body(h, carry2):
            for p in range(2):
                e = 2 * h + p
                a = [slab_ctx[e, pl.ds(dc * 16, 16)] for dc in range(4)]
                for j in range(1, _CTX):
                    r = j * _E + e
                    for dc in range(4):
                        a[dc] = a[dc] + slab_ctx[r, pl.ds(dc * 16, 16)]
                for dc in range(4):
                    a_buf[c * 8 + h, pl.ds(p * 64 + dc * 16, 16)] = a[dc]
            return carry2

        return lax.fori_loop(0, _E // 2, ex_pair_body, carry, unroll=False)

    lax.fori_loop(0, _CHUNKS, chunk_body, 0, unroll=False)
    pltpu.sync_copy(a_buf, a_out.at[pl.ds(wid * (_BPW // 2), _BPW // 2)])


def _sc_word_body(w_word, pw_idx, neg_t, a2, out,
                  idx_neg_o, idx_pw_o, phys_n, phys_p,
                  slab_neg, rows_pw, a_buf, out_v, sem):
    wid = lax.axis_index("s") * _NC + lax.axis_index("c")

    for j in range(_CTX):
        pltpu.sync_copy(neg_t.at[pl.ds(j * _B + wid * _BPW, _BPW)],
                        idx_neg_o.at[j])
    pltpu.sync_copy(pw_idx.at[pl.ds(wid * _BPW, _BPW)], idx_pw_o)
    pltpu.sync_copy(a2.at[pl.ds(wid * (_BPW // 2), _BPW // 2)], a_buf)

    def chunk_body(c, acc):
        e0 = c * _E
        for j in range(_CTX):
            phys_n[pl.ds(j * _E, 16)] = idx_neg_o[j, pl.ds(e0, 16)]
        phys_p[...] = idx_pw_o[pl.ds(e0, 16)]
        copies = []
        for start, size in ((0, 128), (128, 128), (256, 64)):
            copies.append(pltpu.async_copy(
                w_word.at[phys_n.at[pl.ds(start, size)]],
                slab_neg.at[pl.ds(start, size)], sem))
        copies.append(pltpu.async_copy(w_word.at[phys_p], rows_pw, sem))
        for cp in copies:
            cp.wait()

        def ex_pair_body(h, acc2):
            for p in range(2):
                e = 2 * h + p
                a = [a_buf[c * 8 + h, pl.ds(p * 64 + dc * 16, 16)]
                     for dc in range(4)]
                nacc = [slab_neg[e, pl.ds(dc * 16, 16)] for dc in range(4)]
                for j in range(1, _CTX):
                    r = j * _E + e
                    for dc in range(4):
                        nacc[dc] = nacc[dc] + slab_neg[r, pl.ds(dc * 16, 16)]
                pvec = [rows_pw[e, pl.ds(dc * 16, 16)] for dc in range(4)]
                sp = (a[0] * pvec[0] + a[1] * pvec[1] + a[2] * pvec[2] +
                      a[3] * pvec[3])
                sn = (a[0] * nacc[0] + a[1] * nacc[1] + a[2] * nacc[2] +
                      a[3] * nacc[3])
                spl_sp = _lane_sum_splat(sp)
                spl_sn = _lane_sum_splat(sn)
                x = jnp.where(jnp.arange(16, dtype=jnp.int32) < 8,
                              spl_sp, -spl_sn)
                acc2 = acc2 + _logsigmoid(x)
            return acc2

        return lax.fori_loop(0, _E // 2, ex_pair_body, acc, unroll=False)

    accv = lax.fori_loop(0, _CHUNKS, chunk_body,
                         jnp.zeros((16,), jnp.float32), unroll=False)
    total = _lane_sum_splat(accv) * 0.125
    out_v[0, :] = jnp.where(jnp.arange(16, dtype=jnp.int32) == 0, total, 0.0)
    zeros = jnp.zeros((16,), jnp.float32)
    for r in range(1, 8):
        out_v[r, :] = zeros
    pltpu.sync_copy(out_v, out.at[wid])


@jax.jit
def _cbow_loss_sc(w_ctx_p, w_word_p, ctx_t, pw_idx, neg_t):
    mesh = plsc.VectorSubcoreMesh(core_axis_name="c", subcore_axis_name="s")
    ctx_kfn = functools.partial(
        pl.kernel, mesh=mesh,
        out_type=jax.ShapeDtypeStruct((_B // 2, 2 * _EMB_DIM), jnp.float32),
        scratch_types=[
            pltpu.VMEM((_CTX, _BPW), jnp.int32),
            pltpu.VMEM((_CTX * _E,), jnp.int32),
            pltpu.VMEM((_CTX * _E, 2 * _EMB_DIM), jnp.float32),
            pltpu.VMEM((_BPW // 2, 2 * _EMB_DIM), jnp.float32),
            pltpu.SemaphoreType.DMA,
        ],
    )(_sc_ctx_body)
    a2 = ctx_kfn(w_ctx_p, ctx_t)
    word_kfn = functools.partial(
        pl.kernel, mesh=mesh,
        out_type=jax.ShapeDtypeStruct((_NW, 8, 16), jnp.float32),
        scratch_types=[
            pltpu.VMEM((_CTX, _BPW), jnp.int32),
            pltpu.VMEM((_BPW,), jnp.int32),
            pltpu.VMEM((_CTX * _E,), jnp.int32),
            pltpu.VMEM((_E,), jnp.int32),
            pltpu.VMEM((_CTX * _E, 2 * _EMB_DIM), jnp.float32),
            pltpu.VMEM((_E, 2 * _EMB_DIM), jnp.float32),
            pltpu.VMEM((_BPW // 2, 2 * _EMB_DIM), jnp.float32),
            pltpu.VMEM((8, 16), jnp.float32),
            pltpu.SemaphoreType.DMA,
        ],
    )(_sc_word_body)
    return word_kfn(w_word_p, pw_idx, neg_t, a2)


def kernel(W_ctx, W_word, pos_context, pos_word, neg_word):
    proj = jnp.eye(_EMB_DIM, 2 * _EMB_DIM, dtype=jnp.float32)
    w_ctx2 = jnp.dot(W_ctx, proj)
    w_word2 = jnp.dot(W_word, proj)
    ctx_t = pos_context.T.reshape(-1)
    neg_t = neg_word.T.reshape(-1)
    partials = _cbow_loss_sc(w_ctx2, w_word2, ctx_t, pos_word, neg_t)
    return -jnp.sum(partials)


# speedup vs baseline: 1.1306x; 1.0001x over previous
import functools

import jax
import jax.numpy as jnp
from jax import lax
from jax.experimental import pallas as pl
from jax.experimental.pallas import tpu as pltpu
from jax.experimental.pallas import tpu_sc as plsc

_EMB_SIZE = 1000000
_EMB_DIM = 64
_B = 16384
_CTX = 20
_NC = 2
_NS = 16
_NW = _NC * _NS
_BPW = _B // _NW
_E = 32
_CHUNKS = _BPW // _E
_BATCHES = tuple((s, 128) for s in range(0, _CTX * _E, 128))
_E2 = 16
_CHUNKS2 = _BPW // _E2
_BATCHES2 = ((0, 128), (128, 128), (256, 64))

_LN2 = 0.6931471805599453
_SQRT2 = 1.4142135623730951

_DNUMS = lax.GatherDimensionNumbers(
    offset_dims=(), collapsed_slice_dims=(0,), start_index_map=(0,))


def _log_pos(a):
    i = lax.bitcast_convert_type(a, jnp.int32)
    e = lax.shift_right_arithmetic(i, 23) - 127
    m = lax.bitcast_convert_type(
        jnp.bitwise_or(jnp.bitwise_and(i, 0x007FFFFF), 0x3F800000), jnp.float32)
    big = m > _SQRT2
    m = jnp.where(big, m * 0.5, m)
    e = jnp.where(big, e + 1, e)
    t = (m - 1.0) / (m + 1.0)
    t2 = t * t
    series = 1.0 + t2 * (1.0 / 3.0 + t2 * (1.0 / 5.0 + t2 * (
        1.0 / 7.0 + t2 * (1.0 / 9.0 + t2 * (1.0 / 11.0)))))
    return e.astype(jnp.float32) * _LN2 + 2.0 * t * series


def _lane_sum_splat(v):
    idx = jnp.arange(16, dtype=jnp.int32)
    for s in (1, 2, 4, 8):
        perm = jnp.bitwise_xor(idx, s)
        v = v + lax.gather(v, perm[:, None], dimension_numbers=_DNUMS,
                           slice_sizes=(1,),
                           mode=lax.GatherScatterMode.PROMISE_IN_BOUNDS)
    return v


def _logsigmoid(x):
    return -_log_pos(1.0 + jnp.exp(-x))


def _sc_ctx_body(w_ctx, ctx_t, a_out,
                 idx_ctx_o, phys_c, slab_ctx, a_buf, sem):
    wid = lax.axis_index("s") * _NC + lax.axis_index("c")

    for j in range(_CTX):
        pltpu.sync_copy(ctx_t.at[pl.ds(j * _B + wid * _BPW, _BPW)],
                        idx_ctx_o.at[j])

    def chunk_body(c, carry):
        e0 = c * _E
        for j in range(_CTX):
            for g in range(_E // 16):
                phys_c[pl.ds(j * _E + g * 16, 16)] = (
                    idx_ctx_o[j, pl.ds(e0 + g * 16, 16)])
        copies = []
        for start, size in _BATCHES:
            copies.append(pltpu.async_copy(
                w_ctx.at[phys_c.at[pl.ds(start, size)]],
                slab_ctx.at[pl.ds(start, size)], sem))
        for cp in copies:
            cp.wait()

        def ex_pair_body(h, carry2):
            for p in range(2):
                e = 2 * h + p
                a = [slab_ctx[e, pl.ds(dc * 16, 16)] for dc in range(4)]
                for j in range(1, _CTX):
                    r = j * _E + e
                    for dc in range(4):
                        a[dc] = a[dc] + slab_ctx[r, pl.ds(dc * 16, 16)]
                for dc in range(4):
                    a_buf[c * (_E // 2) + h,
                          pl.ds(p * 64 + dc * 16, 16)] = a[dc]
            return carry2

        return lax.fori_loop(0, _E // 2, ex_pair_body, carry, unroll=False)

    lax.fori_loop(0, _CHUNKS, chunk_body, 0, unroll=False)
    pltpu.sync_copy(a_buf, a_out.at[pl.ds(wid * (_BPW // 2), _BPW // 2)])


def _sc_word_body(w_word, pw_idx, neg_t, a2, out,
                  idx_neg_o, idx_pw_o, phys_n, phys_p,
                  slab_neg, rows_pw, a_buf, out_v, sem):
    wid = lax.axis_index("s") * _NC + lax.axis_index("c")

    for j in range(_CTX):
        pltpu.sync_copy(neg_t.at[pl.ds(j * _B + wid * _BPW, _BPW)],
                        idx_neg_o.at[j])
    pltpu.sync_copy(pw_idx.at[pl.ds(wid * _BPW, _BPW)], idx_pw_o)
    pltpu.sync_copy(a2.at[pl.ds(wid * (_BPW // 2), _BPW // 2)], a_buf)

    def chunk_body(c, acc):
        e0 = c * _E2
        for j in range(_CTX):
            phys_n[pl.ds(j * _E2, 16)] = idx_neg_o[j, pl.ds(e0, 16)]
        phys_p[...] = idx_pw_o[pl.ds(e0, 16)]
        copies = []
        for start, size in _BATCHES2:
            copies.append(pltpu.async_copy(
                w_word.at[phys_n.at[pl.ds(start, size)]],
                slab_neg.at[pl.ds(start, size)], sem))
        copies.append(pltpu.async_copy(w_word.at[phys_p], rows_pw, sem))
        for cp in copies:
            cp.wait()

        def ex_pair_body(h, acc2):
            for p in range(2):
                e = 2 * h + p
                a = [a_buf[c * (_E2 // 2) + h, pl.ds(p * 64 + dc * 16, 16)]
                     for dc in range(4)]
                nacc = [slab_neg[e, pl.ds(dc * 16, 16)] for dc in range(4)]
                for j in range(1, _CTX):
                    r = j * _E2 + e
                    for dc in range(4):
                        nacc[dc] = nacc[dc] + slab_neg[r, pl.ds(dc * 16, 16)]
                pvec = [rows_pw[e, pl.ds(dc * 16, 16)] for dc in range(4)]
                sp = (a[0] * pvec[0] + a[1] * pvec[1] + a[2] * pvec[2] +
                      a[3] * pvec[3])
                sn = (a[0] * nacc[0] + a[1] * nacc[1] + a[2] * nacc[2] +
                      a[3] * nacc[3])
                spl_sp = _lane_sum_splat(sp)
                spl_sn = _lane_sum_splat(sn)
                x = jnp.where(jnp.arange(16, dtype=jnp.int32) < 8,
                              spl_sp, -spl_sn)
                acc2 = acc2 + _logsigmoid(x)
            return acc2

        return lax.fori_loop(0, _E2 // 2, ex_pair_body, acc, unroll=False)

    accv = lax.fori_loop(0, _CHUNKS2, chunk_body,
                         jnp.zeros((16,), jnp.float32), unroll=False)
    total = _lane_sum_splat(accv) * 0.125
    out_v[0, :] = jnp.where(jnp.arange(16, dtype=jnp.int32) == 0, total, 0.0)
    zeros = jnp.zeros((16,), jnp.float32)
    for r in range(1, 8):
        out_v[r, :] = zeros
    pltpu.sync_copy(out_v, out.at[wid])


@jax.jit
def _cbow_loss_sc(w_ctx_p, w_word_p, ctx_t, pw_idx, neg_t):
    mesh = plsc.VectorSubcoreMesh(core_axis_name="c", subcore_axis_name="s")
    ctx_kfn = functools.partial(
        pl.kernel, mesh=mesh,
        out_type=jax.ShapeDtypeStruct((_B // 2, 2 * _EMB_DIM), jnp.float32),
        scratch_types=[
            pltpu.VMEM((_CTX, _BPW), jnp.int32),
            pltpu.VMEM((_CTX * _E,), jnp.int32),
            pltpu.VMEM((_CTX * _E, 2 * _EMB_DIM), jnp.float32),
            pltpu.VMEM((_BPW // 2, 2 * _EMB_DIM), jnp.float32),
            pltpu.SemaphoreType.DMA,
        ],
    )(_sc_ctx_body)
    a2 = ctx_kfn(w_ctx_p, ctx_t)
    word_kfn = functools.partial(
        pl.kernel, mesh=mesh,
        out_type=jax.ShapeDtypeStruct((_NW, 8, 16), jnp.float32),
        scratch_types=[
            pltpu.VMEM((_CTX, _BPW), jnp.int32),
            pltpu.VMEM((_BPW,), jnp.int32),
            pltpu.VMEM((_CTX * _E2,), jnp.int32),
            pltpu.VMEM((_E2,), jnp.int32),
            pltpu.VMEM((_CTX * _E2, 2 * _EMB_DIM), jnp.float32),
            pltpu.VMEM((_E2, 2 * _EMB_DIM), jnp.float32),
            pltpu.VMEM((_BPW // 2, 2 * _EMB_DIM), jnp.float32),
            pltpu.VMEM((8, 16), jnp.float32),
            pltpu.SemaphoreType.DMA,
        ],
    )(_sc_word_body)
    return word_kfn(w_word_p, pw_idx, neg_t, a2)


def kernel(W_ctx, W_word, pos_context, pos_word, neg_word):
    proj = jnp.eye(_EMB_DIM, 2 * _EMB_DIM, dtype=jnp.float32)
    w_ctx2 = jnp.dot(W_ctx, proj)
    w_word2 = jnp.dot(W_word, proj)
    ctx_t = pos_context.T.reshape(-1)
    neg_t = neg_word.T.reshape(-1)
    partials = _cbow_loss_sc(w_ctx2, w_word2, ctx_t, pos_word, neg_t)
    return -jnp.sum(partials)
